# Initial kernel scaffold; baseline (speedup 1.0000x reference)
import jax
import jax.numpy as jnp
from jax.experimental import pallas as pl


def kernel(routing_tensor):
    raise NotImplementedError("write your pallas kernel here")


# speedup vs baseline: 1.4827x; 1.4827x over previous
import jax
import jax.numpy as jnp
from jax.experimental import pallas as pl

_TOKENS = 32768
_E = 64
_TAU = 0.25
_BT = 512


def _body(x_ref, mask_ref, routed_ref):
    x = x_ref[...]
    col = jax.lax.broadcasted_iota(jnp.int32, (1, _E), 1)

    s = jnp.zeros((_BT, _E), jnp.float32)
    for j in range(_E):
        xj = x[:, j:j + 1]
        before = (xj > x) | ((xj >= x) & (j < col))
        s = s + jnp.where(before, xj, 0.0)
    m = s < _TAU
    mask_ref[...] = m.astype(jnp.int32)
    routed_ref[...] = jnp.where(m, x, 0.0)


def kernel(routing_tensor):
    grid = (_TOKENS // _BT,)
    spec = pl.BlockSpec((_BT, _E), lambda i: (i, 0))
    mask, routed = pl.pallas_call(
        _body,
        grid=grid,
        in_specs=[spec],
        out_specs=[spec, spec],
        out_shape=[
            jax.ShapeDtypeStruct((_TOKENS, _E), jnp.int32),
            jax.ShapeDtypeStruct((_TOKENS, _E), jnp.float32),
        ],
    )(routing_tensor)
    return (mask, routed)


# speedup vs baseline: 6.4456x; 4.3473x over previous
import functools
import jax
import jax.numpy as jnp
from jax import lax
from jax.experimental import pallas as pl
from jax.experimental.pallas import tpu as pltpu
from jax.experimental.pallas import tpu_sc as plsc

_TOKENS = 32768
_E = 64
_TAU = 0.25
_NW = 32
_TPW = _TOKENS // _NW
_C = 256
_NCHUNK = _TPW // _C


def _rev(v):
    return lax.rev(v, (0,))


def _sortd(v):
    k, _ = plsc.sort_key_val(v, v, descending=True)
    return k


def _sort64_desc(x0, x1, x2, x3):
    s0, s1, s2, s3 = _sortd(x0), _sortd(x1), _sortd(x2), _sortd(x3)
    rb = _rev(s1)
    u1, v1 = _sortd(jnp.maximum(s0, rb)), _sortd(jnp.minimum(s0, rb))
    rb = _rev(s3)
    u2, v2 = _sortd(jnp.maximum(s2, rb)), _sortd(jnp.minimum(s2, rb))
    a, b = _rev(v2), _rev(u2)
    p, r = jnp.maximum(u1, a), jnp.minimum(u1, a)
    q, s = jnp.maximum(v1, b), jnp.minimum(v1, b)
    d0 = _sortd(jnp.maximum(p, q))
    d1 = _sortd(jnp.minimum(p, q))
    d2 = _sortd(jnp.maximum(r, s))
    d3 = _sortd(jnp.minimum(r, s))
    return d0, d1, d2, d3


def _popcnt(b):
    return plsc.all_reduce_population_count(b)


def _token(t, x_v, mask_v, routed_v, s64_v):
    x0 = x_v[t, 0:16]
    x1 = x_v[t, 16:32]
    x2 = x_v[t, 32:48]
    x3 = x_v[t, 48:64]
    d0, d1, d2, d3 = _sort64_desc(x0, x1, x2, x3)

    r0 = jnp.sum(d0)
    r1 = r0 + jnp.sum(d1)
    r2 = r1 + jnp.sum(d2)
    c0 = plsc.cumsum(d0)
    c1 = plsc.cumsum(d1) + r0
    c2 = plsc.cumsum(d2) + r1
    c3 = plsc.cumsum(d3) + r2

    cnt = (_popcnt(c0 < _TAU) + _popcnt(c1 < _TAU)
           + _popcnt(c2 < _TAU) + _popcnt(c3 < _TAU))
    k = jnp.minimum(cnt + 1, _E)

    s64_v[0:16] = d0
    s64_v[16:32] = d1
    s64_v[32:48] = d2
    s64_v[48:64] = d3
    vstar = plsc.load_gather(s64_v, [k - 1])

    g = (_popcnt(x0 > vstar) + _popcnt(x1 > vstar)
         + _popcnt(x2 > vstar) + _popcnt(x3 > vstar))
    m = k - g

    eq0 = x0 == vstar
    eq1 = x1 == vstar
    eq2 = x2 == vstar
    eq3 = x3 == vstar
    n0 = _popcnt(eq0)
    n1 = n0 + _popcnt(eq1)
    n2 = n1 + _popcnt(eq2)
    cc0 = plsc.cumsum(eq0.astype(jnp.int32))
    cc1 = plsc.cumsum(eq1.astype(jnp.int32)) + n0
    cc2 = plsc.cumsum(eq2.astype(jnp.int32)) + n1
    cc3 = plsc.cumsum(eq3.astype(jnp.int32)) + n2

    sel0 = (x0 > vstar) | (eq0 & (cc0 <= m))
    sel1 = (x1 > vstar) | (eq1 & (cc1 <= m))
    sel2 = (x2 > vstar) | (eq2 & (cc2 <= m))
    sel3 = (x3 > vstar) | (eq3 & (cc3 <= m))

    mask_v[t, 0:16] = sel0.astype(jnp.int32)
    mask_v[t, 16:32] = sel1.astype(jnp.int32)
    mask_v[t, 32:48] = sel2.astype(jnp.int32)
    mask_v[t, 48:64] = sel3.astype(jnp.int32)
    routed_v[t, 0:16] = jnp.where(sel0, x0, 0.0)
    routed_v[t, 16:32] = jnp.where(sel1, x1, 0.0)
    routed_v[t, 32:48] = jnp.where(sel2, x2, 0.0)
    routed_v[t, 48:64] = jnp.where(sel3, x3, 0.0)


def _sc_body(x_hbm, mask_hbm, routed_hbm, x_v, mask_v, routed_v, s64_v):
    wid = lax.axis_index("s") * 2 + lax.axis_index("c")
    base = wid * _TPW

    def chunk(i, carry):
        tok0 = base + i * _C
        pltpu.sync_copy(x_hbm.at[pl.ds(tok0, _C)], x_v)

        def tok(t, c2):
            _token(t, x_v, mask_v, routed_v, s64_v)
            return c2

        lax.fori_loop(0, _C, tok, 0)
        pltpu.sync_copy(mask_v, mask_hbm.at[pl.ds(tok0, _C)])
        pltpu.sync_copy(routed_v, routed_hbm.at[pl.ds(tok0, _C)])
        return carry

    lax.fori_loop(0, _NCHUNK, chunk, 0)


_sc_kernel = functools.partial(
    pl.kernel,
    out_type=[
        jax.ShapeDtypeStruct((_TOKENS, _E), jnp.int32),
        jax.ShapeDtypeStruct((_TOKENS, _E), jnp.float32),
    ],
    mesh=plsc.VectorSubcoreMesh(core_axis_name="c", subcore_axis_name="s"),
    scratch_types=[
        pltpu.VMEM((_C, _E), jnp.float32),
        pltpu.VMEM((_C, _E), jnp.int32),
        pltpu.VMEM((_C, _E), jnp.float32),
        pltpu.VMEM((_E,), jnp.float32),
    ],
    compiler_params=pltpu.CompilerParams(needs_layout_passes=False),
)(_sc_body)


def kernel(routing_tensor):
    mask, routed = _sc_kernel(routing_tensor)
    return (mask, routed)


# speedup vs baseline: 6.7980x; 1.0547x over previous
import functools
import jax
import jax.numpy as jnp
from jax import lax
from jax.experimental import pallas as pl
from jax.experimental.pallas import tpu as pltpu
from jax.experimental.pallas import tpu_sc as plsc

_TOKENS = 32768
_E = 64
_TAU = 0.25
_NW = 32
_TPW = _TOKENS // _NW
_C = 256
_NCHUNK = _TPW // _C


def _rev(v):
    return lax.rev(v, (0,))


def _sortd(v):
    k, _ = plsc.sort_key_val(v, v, descending=True)
    return k


def _sort64_desc(x0, x1, x2, x3):
    s0, s1, s2, s3 = _sortd(x0), _sortd(x1), _sortd(x2), _sortd(x3)
    rb = _rev(s1)
    u1, v1 = _sortd(jnp.maximum(s0, rb)), _sortd(jnp.minimum(s0, rb))
    rb = _rev(s3)
    u2, v2 = _sortd(jnp.maximum(s2, rb)), _sortd(jnp.minimum(s2, rb))
    a, b = _rev(v2), _rev(u2)
    p, r = jnp.maximum(u1, a), jnp.minimum(u1, a)
    q, s = jnp.maximum(v1, b), jnp.minimum(v1, b)
    d0 = _sortd(jnp.maximum(p, q))
    d1 = _sortd(jnp.minimum(p, q))
    d2 = _sortd(jnp.maximum(r, s))
    d3 = _sortd(jnp.minimum(r, s))
    return d0, d1, d2, d3


def _popcnt(b):
    return plsc.all_reduce_population_count(b)


def _first_eq_mask(x, vstar):
    eq = [xi == vstar for xi in x]
    n0 = _popcnt(eq[0])
    n1 = n0 + _popcnt(eq[1])
    n2 = n1 + _popcnt(eq[2])
    cc0 = plsc.cumsum(eq[0].astype(jnp.int32))
    cc1 = plsc.cumsum(eq[1].astype(jnp.int32)) + n0
    cc2 = plsc.cumsum(eq[2].astype(jnp.int32)) + n1
    cc3 = plsc.cumsum(eq[3].astype(jnp.int32)) + n2
    return eq, (cc0, cc1, cc2, cc3)


def _token_masks(x0, x1, x2, x3):
    m = jnp.max(jnp.maximum(jnp.maximum(x0, x1), jnp.maximum(x2, x3)))

    def fast(_):
        vstar = jnp.zeros((16,), jnp.float32) + m
        eq, cc = _first_eq_mask((x0, x1, x2, x3), vstar)
        return tuple(e & (c == 1) for e, c in zip(eq, cc))

    def slow(_):
        d0, d1, d2, d3 = _sort64_desc(x0, x1, x2, x3)
        r0 = jnp.sum(d0)
        r1 = r0 + jnp.sum(d1)
        r2 = r1 + jnp.sum(d2)
        c0 = plsc.cumsum(d0)
        c1 = plsc.cumsum(d1) + r0
        c2 = plsc.cumsum(d2) + r1
        c3 = plsc.cumsum(d3) + r2
        cnt = (_popcnt(c0 < _TAU) + _popcnt(c1 < _TAU)
               + _popcnt(c2 < _TAU) + _popcnt(c3 < _TAU))
        k = jnp.minimum(cnt + 1, _E)
        km1 = k - 1
        iot = lax.iota(jnp.int32, 16)
        pick = (jnp.where(iot == km1, d0, 0.0)
                + jnp.where(iot + 16 == km1, d1, 0.0)
                + jnp.where(iot + 32 == km1, d2, 0.0)
                + jnp.where(iot + 48 == km1, d3, 0.0))
        vstar = jnp.zeros((16,), jnp.float32) + jnp.sum(pick)
        gt = [x0 > vstar, x1 > vstar, x2 > vstar, x3 > vstar]
        g = _popcnt(gt[0]) + _popcnt(gt[1]) + _popcnt(gt[2]) + _popcnt(gt[3])
        mm = k - g
        eq, cc = _first_eq_mask((x0, x1, x2, x3), vstar)
        return tuple(gti | (e & (c <= mm))
                     for gti, e, c in zip(gt, eq, cc))

    return lax.cond(m >= _TAU, fast, slow, 0)


def _sc_body(x_hbm, mask_hbm, routed_hbm, x_v, mask_v, routed_v):
    wid = lax.axis_index("s") * 2 + lax.axis_index("c")
    base = wid * _TPW

    def chunk(i, carry):
        tok0 = base + i * _C
        pltpu.sync_copy(x_hbm.at[pl.ds(tok0, _C)], x_v)

        @plsc.parallel_loop(0, _C, unroll=4)
        def tok(t):
            x0 = x_v[t, 0:16]
            x1 = x_v[t, 16:32]
            x2 = x_v[t, 32:48]
            x3 = x_v[t, 48:64]
            sel0, sel1, sel2, sel3 = _token_masks(x0, x1, x2, x3)
            mask_v[t, 0:16] = sel0.astype(jnp.int32)
            mask_v[t, 16:32] = sel1.astype(jnp.int32)
            mask_v[t, 32:48] = sel2.astype(jnp.int32)
            mask_v[t, 48:64] = sel3.astype(jnp.int32)
            routed_v[t, 0:16] = jnp.where(sel0, x0, 0.0)
            routed_v[t, 16:32] = jnp.where(sel1, x1, 0.0)
            routed_v[t, 32:48] = jnp.where(sel2, x2, 0.0)
            routed_v[t, 48:64] = jnp.where(sel3, x3, 0.0)

        pltpu.sync_copy(mask_v, mask_hbm.at[pl.ds(tok0, _C)])
        pltpu.sync_copy(routed_v, routed_hbm.at[pl.ds(tok0, _C)])
        return carry

    lax.fori_loop(0, _NCHUNK, chunk, 0)


_sc_kernel = functools.partial(
    pl.kernel,
    out_type=[
        jax.ShapeDtypeStruct((_TOKENS, _E), jnp.int32),
        jax.ShapeDtypeStruct((_TOKENS, _E), jnp.float32),
    ],
    mesh=plsc.VectorSubcoreMesh(core_axis_name="c", subcore_axis_name="s"),
    scratch_types=[
        pltpu.VMEM((_C, _E), jnp.float32),
        pltpu.VMEM((_C, _E), jnp.int32),
        pltpu.VMEM((_C, _E), jnp.float32),
    ],
    compiler_params=pltpu.CompilerParams(needs_layout_passes=False),
)(_sc_body)


def kernel(routing_tensor):
    mask, routed = _sc_kernel(routing_tensor)
    return (mask, routed)
